# baseline (speedup 1.0000x reference)
import functools

import jax
import jax.numpy as jnp
from jax import lax
from jax.experimental import pallas as pl
from jax.experimental.pallas import tpu as pltpu
from jax.experimental.pallas import tpu_sc as plsc

_N = 50000
_E = 800000
_D = 64
_DH = 32
_G = 64

_NC = 2
_NS = 16
_ZSLICE = 3128
_ACC_ROWS = _NS * _ZSLICE
_LASTSLICE = _N - (_NS - 1) * _ZSLICE
_DRB = 248

_CHUNK = 128
_NCHUNK = _E // _CHUNK
_CPS = -(-_NCHUNK // _NS)

_PREC = lax.Precision.HIGHEST


def _dot(a, b):
    return lax.dot_general(a, b, (((1,), (0,)), ((), ())),
                           precision=_PREC, preferred_element_type=jnp.float32)


_BE = 2000


def _emlp_body(a_ref, w1_ref, b1_ref, w2_ref, b2_ref, o_ref):
    t = jnp.maximum(_dot(a_ref[...], w1_ref[...]) + b1_ref[...], 0.0)
    o_ref[0] = _dot(t, w2_ref[0]) + b2_ref[0]


def _edge_mlp(edge_attr, w1, b1, w2h, b2h):
    return pl.pallas_call(
        _emlp_body,
        grid=(2, _E // _BE),
        in_specs=[
            pl.BlockSpec((_BE, 16), lambda c, i: (i, 0)),
            pl.BlockSpec((16, _D), lambda c, i: (0, 0)),
            pl.BlockSpec((1, _D), lambda c, i: (0, 0)),
            pl.BlockSpec((1, _D, _DH), lambda c, i: (c, 0, 0)),
            pl.BlockSpec((1, 1, _DH), lambda c, i: (c, 0, 0)),
        ],
        out_specs=pl.BlockSpec((1, _BE, _DH), lambda c, i: (c, i, 0)),
        out_shape=jax.ShapeDtypeStruct((2, _E, _DH), jnp.float32),
    )(edge_attr, w1, b1, w2h, b2h)


_BN = 2000


def _pool_body(h_ref, b_ref, wl_ref, bl_ref, o_ref, acc_ref):
    c = pl.program_id(0)
    i = pl.program_id(1)

    @pl.when(jnp.logical_and(c == 0, i == 0))
    def _():
        acc_ref[...] = jnp.zeros_like(acc_ref)

    seg = b_ref[0]
    onehot = (lax.broadcasted_iota(jnp.int32, (_G, _BN), 0) == seg
              ).astype(jnp.float32)
    half = _dot(onehot, h_ref[0])
    acc_ref[...] += _dot(half, wl_ref[0])

    @pl.when(jnp.logical_and(c == pl.num_programs(0) - 1,
                             i == pl.num_programs(1) - 1))
    def _():
        o_ref[...] = acc_ref[...] + bl_ref[...]


def _pool(h2, batch3d, wl2, bl):
    return pl.pallas_call(
        _pool_body,
        grid=(2, _N // _BN),
        in_specs=[
            pl.BlockSpec((1, _BN, _DH), lambda c, i: (c, i, 0)),
            pl.BlockSpec((1, 1, _BN), lambda c, i: (i, 0, 0)),
            pl.BlockSpec((1, _DH, 1), lambda c, i: (c, 0, 0)),
            pl.BlockSpec((1, 1), lambda c, i: (0, 0)),
        ],
        out_specs=pl.BlockSpec((_G, 1), lambda c, i: (0, 0)),
        out_shape=jax.ShapeDtypeStruct((_G, 1), jnp.float32),
        scratch_shapes=[pltpu.VMEM((_G, 1), jnp.float32)],
    )(h2, batch3d, wl2, bl)


def _sc_body(x2_hbm, src_hbm, dst_hbm, e0_hbm, e1_hbm, e2_hbm, z_hbm,
             out_hbm, sidx, didx, rows, ev, accum, sems):
    cid = lax.axis_index("c")
    sid = lax.axis_index("s")
    cbase = sid * _CPS

    h0 = x2_hbm.at[cid]
    hcur = out_hbm.at[cid]

    def run_layer(h_view, e2_hbm_l):
        e_view = e2_hbm_l.at[cid]

        zlo = sid * _ZSLICE
        pltpu.sync_copy(z_hbm.at[pl.ds(zlo, _ZSLICE)],
                        accum.at[pl.ds(zlo, _ZSLICE)])
        plsc.subcore_barrier()

        def valid(k):
            return jnp.logical_and(k < _CPS, cbase + k < _NCHUNK)

        def idx_start(k):
            @pl.when(valid(k))
            def _():
                b = lax.rem(k, 2)
                ebase = pl.multiple_of((cbase + k) * _CHUNK, _CHUNK)
                pltpu.async_copy(src_hbm.at[pl.ds(ebase, _CHUNK)],
                                 sidx.at[b], sems.at[b])
                pltpu.async_copy(dst_hbm.at[pl.ds(ebase, _CHUNK)],
                                 didx.at[b], sems.at[b])

        def gather_start(k):
            @pl.when(valid(k))
            def _():
                b = lax.rem(k, 2)
                boff = pl.multiple_of(b * _CHUNK, _CHUNK)
                pltpu.make_async_copy(src_hbm.at[pl.ds(0, _CHUNK)],
                                      sidx.at[b], sems.at[b]).wait()
                pltpu.make_async_copy(dst_hbm.at[pl.ds(0, _CHUNK)],
                                      didx.at[b], sems.at[b]).wait()
                pltpu.async_copy(h_view.at[sidx.at[b]],
                                 rows.at[pl.ds(boff, _CHUNK)], sems.at[b])
                ebase = pl.multiple_of((cbase + k) * _CHUNK, _CHUNK)
                pltpu.async_copy(e_view.at[pl.ds(ebase, _CHUNK)],
                                 ev.at[pl.ds(boff, _CHUNK)], sems.at[b])

        def process(k):
            @pl.when(valid(k))
            def _():
                b = lax.rem(k, 2)
                boff = pl.multiple_of(b * _CHUNK, _CHUNK)
                pltpu.make_async_copy(h_view.at[sidx.at[b]],
                                      rows.at[pl.ds(boff, _CHUNK)],
                                      sems.at[b]).wait()
                pltpu.make_async_copy(e_view.at[pl.ds(0, _CHUNK)],
                                      ev.at[pl.ds(boff, _CHUNK)],
                                      sems.at[b]).wait()

                @pl.loop(0, _CHUNK, unroll=4)
                def _(r):
                    for q in range(_DH // 16):
                        sl = (boff + r, pl.ds(q * 16, 16))
                        rows[sl] = jnp.maximum(rows[sl] + ev[sl], 0.0)

                pltpu.sync_copy(rows.at[pl.ds(boff, _CHUNK)],
                                accum.at[didx.at[b]], add=True)

        idx_start(0)
        idx_start(1)
        gather_start(0)

        @pl.loop(0, _CPS)
        def _(k):
            gather_start(k + 1)
            process(k)
            idx_start(k + 2)

        plsc.subcore_barrier()

        nlo = sid * _ZSLICE
        nblocks = _ZSLICE // _DRB
        rem = _ZSLICE - nblocks * _DRB
        is_last = sid == _NS - 1
        lastrem = _LASTSLICE - nblocks * _DRB

        def drain_block(off, size):
            pltpu.sync_copy(h_view.at[pl.ds(nlo + off, size)],
                            rows.at[pl.ds(0, size)])
            pltpu.sync_copy(accum.at[pl.ds(nlo + off, size)],
                            ev.at[pl.ds(0, size)])

            @pl.loop(0, size)
            def _(r):
                for q in range(_DH // 16):
                    sl = (r, pl.ds(q * 16, 16))
                    rows[sl] = jnp.maximum(rows[sl] + ev[sl], 0.0)

            pltpu.sync_copy(rows.at[pl.ds(0, size)],
                            hcur.at[pl.ds(nlo + off, size)])

        @pl.loop(0, nblocks)
        def _(j):
            drain_block(j * _DRB, _DRB)

        @pl.when(jnp.logical_not(is_last))
        def _():
            drain_block(nblocks * _DRB, rem)

        @pl.when(is_last)
        def _():
            drain_block(nblocks * _DRB, lastrem)

        plsc.subcore_barrier()

    run_layer(h0, e0_hbm)
    run_layer(hcur, e1_hbm)
    run_layer(hcur, e2_hbm)


@functools.cache
def _sc_pass_fn():
    mesh = plsc.VectorSubcoreMesh(core_axis_name="c", subcore_axis_name="s",
                                  num_cores=_NC, num_subcores=_NS)
    return pl.kernel(
        _sc_body,
        out_type=jax.ShapeDtypeStruct((2, _N, _DH), jnp.float32),
        mesh=mesh,
        scratch_types=[
            pltpu.VMEM((2, _CHUNK), jnp.int32),
            pltpu.VMEM((2, _CHUNK), jnp.int32),
            pltpu.VMEM((2 * _CHUNK, _DH), jnp.float32),
            pltpu.VMEM((2 * _CHUNK, _DH), jnp.float32),
            pltpu.VMEM_SHARED((_ACC_ROWS, _DH), jnp.float32),
            pltpu.SemaphoreType.DMA((2,)),
        ],
        compiler_params=pltpu.CompilerParams(use_tc_tiling_on_sc=False),
    )


def kernel(x, edge_index, edge_attr, batch,
           W1_0, b1_0, W2_0, b2_0,
           W1_1, b1_1, W2_1, b2_1,
           W1_2, b1_2, W2_2, b2_2,
           Wl, bl):
    src = edge_index[0]
    dst = edge_index[1]
    zeros = jnp.zeros((_ACC_ROWS, _DH), jnp.float32)
    x2 = jnp.stack([x[:, :_DH], x[:, _DH:]])
    wl2 = Wl.reshape(2, _DH, 1)

    def _halves(w2, b2):
        return (jnp.stack([w2[:, :_DH], w2[:, _DH:]]),
                b2.reshape(2, 1, _DH))

    e0 = _edge_mlp(edge_attr, W1_0, b1_0.reshape(1, -1), *_halves(W2_0, b2_0))
    e1 = _edge_mlp(edge_attr, W1_1, b1_1.reshape(1, -1), *_halves(W2_1, b2_1))
    e2 = _edge_mlp(edge_attr, W1_2, b1_2.reshape(1, -1), *_halves(W2_2, b2_2))

    h2 = _sc_pass_fn()(x2, src, dst, e0, e1, e2, zeros)

    out = _pool(h2, batch.reshape(_N // _BN, 1, _BN), wl2, bl.reshape(1, 1))
    return jnp.squeeze(out, -1)


# speedup vs baseline: 1.4802x; 1.4802x over previous
import functools

import jax
import jax.numpy as jnp
from jax import lax
from jax.experimental import pallas as pl
from jax.experimental.pallas import tpu as pltpu
from jax.experimental.pallas import tpu_sc as plsc

_N = 50000
_E = 800000
_D = 64
_G = 64

_NC = 2
_NS = 16
_HALF = _N // _NC
_TRASH = _HALF
_ZSLICE = 1568
_SPM_ROWS = _NS * _ZSLICE
_LASTSLICE = _HALF - (_NS - 1) * _ZSLICE

_CHUNK = 128
_NCHUNK = _E // _CHUNK
_CPS = -(-_NCHUNK // _NS)

_PREC = lax.Precision.HIGHEST


def _dot(a, b):
    return lax.dot_general(a, b, (((1,), (0,)), ((), ())),
                           precision=_PREC, preferred_element_type=jnp.float32)


_BE = 2000


def _emlp_body(a_ref, w1_ref, b1_ref, w2_ref, b2_ref, o_ref):
    t = jnp.maximum(_dot(a_ref[...], w1_ref[...]) + b1_ref[...], 0.0)
    o_ref[...] = _dot(t, w2_ref[...]) + b2_ref[...]


def _edge_mlp(edge_attr, w1, b1, w2, b2):
    grid = _E // _BE
    return pl.pallas_call(
        _emlp_body,
        grid=(grid,),
        in_specs=[
            pl.BlockSpec((_BE, 16), lambda i: (i, 0)),
            pl.BlockSpec((16, _D), lambda i: (0, 0)),
            pl.BlockSpec((1, _D), lambda i: (0, 0)),
            pl.BlockSpec((_D, _D), lambda i: (0, 0)),
            pl.BlockSpec((1, _D), lambda i: (0, 0)),
        ],
        out_specs=pl.BlockSpec((_BE, _D), lambda i: (i, 0)),
        out_shape=jax.ShapeDtypeStruct((_E, _D), jnp.float32),
    )(edge_attr, w1, b1, w2, b2)


_BN = 2000


def _update_body(h_ref, a_ref, o_ref):
    o_ref[...] = jnp.maximum(h_ref[...] + a_ref[...], 0.0)


def _update(h, agg):
    grid = _N // _BN
    return pl.pallas_call(
        _update_body,
        grid=(grid,),
        in_specs=[
            pl.BlockSpec((_BN, _D), lambda i: (i, 0)),
            pl.BlockSpec((_BN, _D), lambda i: (i, 0)),
        ],
        out_specs=pl.BlockSpec((_BN, _D), lambda i: (i, 0)),
        out_shape=jax.ShapeDtypeStruct((_N, _D), jnp.float32),
    )(h, agg)


def _pool_body(h_ref, b_ref, wl_ref, bl_ref, o_ref, acc_ref):
    i = pl.program_id(0)

    @pl.when(i == 0)
    def _():
        acc_ref[...] = jnp.zeros_like(acc_ref)

    seg = b_ref[0]
    onehot = (lax.broadcasted_iota(jnp.int32, (_G, _BN), 0) == seg
              ).astype(jnp.float32)
    acc_ref[...] += _dot(onehot, h_ref[...])

    @pl.when(i == pl.num_programs(0) - 1)
    def _():
        o_ref[...] = _dot(acc_ref[...], wl_ref[...]) + bl_ref[...]


def _pool(h, batch3d, wl, bl):
    grid = _N // _BN
    return pl.pallas_call(
        _pool_body,
        grid=(grid,),
        in_specs=[
            pl.BlockSpec((_BN, _D), lambda i: (i, 0)),
            pl.BlockSpec((1, 1, _BN), lambda i: (i, 0, 0)),
            pl.BlockSpec((_D, 1), lambda i: (0, 0)),
            pl.BlockSpec((1, 1), lambda i: (0, 0)),
        ],
        out_specs=pl.BlockSpec((_G, 1), lambda i: (0, 0)),
        out_shape=jax.ShapeDtypeStruct((_G, 1), jnp.float32),
        scratch_shapes=[pltpu.VMEM((_G, _D), jnp.float32)],
    )(h, batch3d, wl, bl)


def _sc_body(h_hbm, src_hbm, dst_hbm, e_hbm, z_hbm, out_hbm,
             sidx, didx, rows, ev, accum, sem_g, sem_e):
    cid = lax.axis_index("c")
    sid = lax.axis_index("s")
    base_node = cid * _HALF

    zlo = sid * _ZSLICE
    pltpu.sync_copy(z_hbm.at[pl.ds(zlo, _ZSLICE)], accum.at[pl.ds(zlo, _ZSLICE)])
    plsc.subcore_barrier()

    @pl.loop(0, _CPS)
    def _(i):
        ck = sid * _CPS + i

        @pl.when(ck < _NCHUNK)
        def _():
            ebase = pl.multiple_of(ck * _CHUNK, _CHUNK)
            pltpu.sync_copy(src_hbm.at[pl.ds(ebase, _CHUNK)], sidx)
            pltpu.sync_copy(dst_hbm.at[pl.ds(ebase, _CHUNK)], didx.at[0])
            cg = pltpu.async_copy(h_hbm.at[sidx], rows, sem_g)
            ce = pltpu.async_copy(e_hbm.at[pl.ds(ebase, _CHUNK)], ev, sem_e)

            @pl.loop(0, _CHUNK // 16)
            def _(j):
                v = didx[0, pl.ds(j * 16, 16)]
                lo = v - base_node
                ok = (lo >= 0) & (lo < _HALF)
                didx[0, pl.ds(j * 16, 16)] = jnp.where(ok, lo, _TRASH)

            cg.wait()
            ce.wait()

            @pl.loop(0, _CHUNK, unroll=4)
            def _(r):
                for q in range(4):
                    sl = (r, pl.ds(q * 16, 16))
                    rows[sl] = jnp.maximum(rows[sl] + ev[sl], 0.0)

            pltpu.sync_copy(rows, accum.at[didx.at[0]], add=True)

    plsc.subcore_barrier()

    @pl.when(sid < _NS - 1)
    def _():
        pltpu.sync_copy(accum.at[pl.ds(sid * _ZSLICE, _ZSLICE)],
                        out_hbm.at[pl.ds(base_node + sid * _ZSLICE, _ZSLICE)])

    @pl.when(sid == _NS - 1)
    def _():
        pltpu.sync_copy(
            accum.at[pl.ds((_NS - 1) * _ZSLICE, _LASTSLICE)],
            out_hbm.at[pl.ds(base_node + (_NS - 1) * _ZSLICE, _LASTSLICE)])


@functools.cache
def _sc_pass_fn():
    mesh = plsc.VectorSubcoreMesh(core_axis_name="c", subcore_axis_name="s",
                                  num_cores=_NC, num_subcores=_NS)
    return pl.kernel(
        _sc_body,
        out_type=jax.ShapeDtypeStruct((_N, _D), jnp.float32),
        mesh=mesh,
        scratch_types=[
            pltpu.VMEM((_CHUNK,), jnp.int32),
            pltpu.VMEM((1, _CHUNK), jnp.int32),
            pltpu.VMEM((_CHUNK, _D), jnp.float32),
            pltpu.VMEM((_CHUNK, _D), jnp.float32),
            pltpu.VMEM_SHARED((_SPM_ROWS, _D), jnp.float32),
            pltpu.SemaphoreType.DMA,
            pltpu.SemaphoreType.DMA,
        ],
        compiler_params=pltpu.CompilerParams(use_tc_tiling_on_sc=False),
    )


def kernel(x, edge_index, edge_attr, batch,
           W1_0, b1_0, W2_0, b2_0,
           W1_1, b1_1, W2_1, b2_1,
           W1_2, b1_2, W2_2, b2_2,
           Wl, bl):
    src = edge_index[0]
    dst = edge_index[1]
    zeros = jnp.zeros((_SPM_ROWS, _D), jnp.float32)

    es = [
        _edge_mlp(edge_attr, W1_0, b1_0.reshape(1, -1), W2_0, b2_0.reshape(1, -1)),
        _edge_mlp(edge_attr, W1_1, b1_1.reshape(1, -1), W2_1, b2_1.reshape(1, -1)),
        _edge_mlp(edge_attr, W1_2, b1_2.reshape(1, -1), W2_2, b2_2.reshape(1, -1)),
    ]

    h = x
    for e in es:
        agg = _sc_pass_fn()(h, src, dst, e, zeros)
        h = _update(h, agg)

    out = _pool(h, batch.reshape(_N // _BN, 1, _BN), Wl, bl.reshape(1, 1))
    return jnp.squeeze(out, -1)


# speedup vs baseline: 1.8670x; 1.2613x over previous
import functools

import jax
import jax.numpy as jnp
from jax import lax
from jax.experimental import pallas as pl
from jax.experimental.pallas import tpu as pltpu
from jax.experimental.pallas import tpu_sc as plsc

_N = 50000
_E = 800000
_D = 64
_G = 64

_NC = 2
_NS = 16
_HALF = _N // _NC
_TRASH = _HALF
_ZSLICE = 1568
_SPM_ROWS = _NS * _ZSLICE
_LASTSLICE = _HALF - (_NS - 1) * _ZSLICE

_CHUNK = 128
_NCHUNK = _E // _CHUNK
_CPS = -(-_NCHUNK // _NS)

_PREC = lax.Precision.HIGHEST


def _dot(a, b):
    return lax.dot_general(a, b, (((1,), (0,)), ((), ())),
                           precision=_PREC, preferred_element_type=jnp.float32)


_BE = 2000


def _emlp_body(a_ref, w1_ref, b1_ref, w2_ref, b2_ref, o_ref):
    t = jnp.maximum(_dot(a_ref[...], w1_ref[...]) + b1_ref[...], 0.0)
    o_ref[...] = _dot(t, w2_ref[...]) + b2_ref[...]


def _edge_mlp(edge_attr, w1, b1, w2, b2):
    grid = _E // _BE
    return pl.pallas_call(
        _emlp_body,
        grid=(grid,),
        in_specs=[
            pl.BlockSpec((_BE, 16), lambda i: (i, 0)),
            pl.BlockSpec((16, _D), lambda i: (0, 0)),
            pl.BlockSpec((1, _D), lambda i: (0, 0)),
            pl.BlockSpec((_D, _D), lambda i: (0, 0)),
            pl.BlockSpec((1, _D), lambda i: (0, 0)),
        ],
        out_specs=pl.BlockSpec((_BE, _D), lambda i: (i, 0)),
        out_shape=jax.ShapeDtypeStruct((_E, _D), jnp.float32),
    )(edge_attr, w1, b1, w2, b2)


_BN = 2000


def _update_body(h_ref, a_ref, o_ref):
    o_ref[...] = jnp.maximum(h_ref[...] + a_ref[...], 0.0)


def _update(h, agg):
    grid = _N // _BN
    return pl.pallas_call(
        _update_body,
        grid=(grid,),
        in_specs=[
            pl.BlockSpec((_BN, _D), lambda i: (i, 0)),
            pl.BlockSpec((_BN, _D), lambda i: (i, 0)),
        ],
        out_specs=pl.BlockSpec((_BN, _D), lambda i: (i, 0)),
        out_shape=jax.ShapeDtypeStruct((_N, _D), jnp.float32),
    )(h, agg)


def _pool_body(h_ref, b_ref, wl_ref, bl_ref, o_ref, acc_ref):
    i = pl.program_id(0)

    @pl.when(i == 0)
    def _():
        acc_ref[...] = jnp.zeros_like(acc_ref)

    seg = b_ref[0]
    onehot = (lax.broadcasted_iota(jnp.int32, (_G, _BN), 0) == seg
              ).astype(jnp.float32)
    acc_ref[...] += _dot(onehot, h_ref[...])

    @pl.when(i == pl.num_programs(0) - 1)
    def _():
        o_ref[...] = _dot(acc_ref[...], wl_ref[...]) + bl_ref[...]


def _pool(h, batch3d, wl, bl):
    grid = _N // _BN
    return pl.pallas_call(
        _pool_body,
        grid=(grid,),
        in_specs=[
            pl.BlockSpec((_BN, _D), lambda i: (i, 0)),
            pl.BlockSpec((1, 1, _BN), lambda i: (i, 0, 0)),
            pl.BlockSpec((_D, 1), lambda i: (0, 0)),
            pl.BlockSpec((1, 1), lambda i: (0, 0)),
        ],
        out_specs=pl.BlockSpec((_G, 1), lambda i: (0, 0)),
        out_shape=jax.ShapeDtypeStruct((_G, 1), jnp.float32),
        scratch_shapes=[pltpu.VMEM((_G, _D), jnp.float32)],
    )(h, batch3d, wl, bl)


def _sc_body(h_hbm, src_hbm, dst_hbm, e_hbm, z_hbm, out_hbm,
             sidx, didx, rows, ev, accum, sem_g, sem_e):
    cid = lax.axis_index("c")
    sid = lax.axis_index("s")
    base_node = cid * _HALF

    zlo = sid * _ZSLICE
    pltpu.sync_copy(z_hbm.at[pl.ds(zlo, _ZSLICE)], accum.at[pl.ds(zlo, _ZSLICE)])
    plsc.subcore_barrier()

    @pl.loop(0, _CPS)
    def _(i):
        ck = sid * _CPS + i

        @pl.when(ck < _NCHUNK)
        def _():
            ebase = pl.multiple_of(ck * _CHUNK, _CHUNK)
            pltpu.sync_copy(src_hbm.at[pl.ds(ebase, _CHUNK)], sidx)
            pltpu.sync_copy(dst_hbm.at[pl.ds(ebase, _CHUNK)], didx.at[0])
            cg = pltpu.async_copy(h_hbm.at[sidx], rows, sem_g)
            ce = pltpu.async_copy(e_hbm.at[pl.ds(ebase, _CHUNK)], ev, sem_e)

            @pl.loop(0, _CHUNK // 16)
            def _(j):
                v = didx[0, pl.ds(j * 16, 16)]
                lo = v - base_node
                ok = (lo >= 0) & (lo < _HALF)
                didx[0, pl.ds(j * 16, 16)] = jnp.where(ok, lo, _TRASH)

            cg.wait()
            ce.wait()

            @pl.loop(0, _CHUNK)
            def _(r):
                for q in range(4):
                    sl = (r, pl.ds(q * 16, 16))
                    rows[sl] = jnp.maximum(rows[sl] + ev[sl], 0.0)

            pltpu.sync_copy(rows, accum.at[didx.at[0]], add=True)

    plsc.subcore_barrier()

    @pl.when(sid < _NS - 1)
    def _():
        pltpu.sync_copy(accum.at[pl.ds(sid * _ZSLICE, _ZSLICE)],
                        out_hbm.at[pl.ds(base_node + sid * _ZSLICE, _ZSLICE)])

    @pl.when(sid == _NS - 1)
    def _():
        pltpu.sync_copy(
            accum.at[pl.ds((_NS - 1) * _ZSLICE, _LASTSLICE)],
            out_hbm.at[pl.ds(base_node + (_NS - 1) * _ZSLICE, _LASTSLICE)])


@functools.cache
def _sc_pass_fn():
    mesh = plsc.VectorSubcoreMesh(core_axis_name="c", subcore_axis_name="s",
                                  num_cores=_NC, num_subcores=_NS)
    return pl.kernel(
        _sc_body,
        out_type=jax.ShapeDtypeStruct((_N, _D), jnp.float32),
        mesh=mesh,
        scratch_types=[
            pltpu.VMEM((_CHUNK,), jnp.int32),
            pltpu.VMEM((1, _CHUNK), jnp.int32),
            pltpu.VMEM((_CHUNK, _D), jnp.float32),
            pltpu.VMEM((_CHUNK, _D), jnp.float32),
            pltpu.VMEM_SHARED((_SPM_ROWS, _D), jnp.float32),
            pltpu.SemaphoreType.DMA,
            pltpu.SemaphoreType.DMA,
        ],
        compiler_params=pltpu.CompilerParams(use_tc_tiling_on_sc=False),
    )


def kernel(x, edge_index, edge_attr, batch,
           W1_0, b1_0, W2_0, b2_0,
           W1_1, b1_1, W2_1, b2_1,
           W1_2, b1_2, W2_2, b2_2,
           Wl, bl):
    src = edge_index[0]
    dst = edge_index[1]
    zeros = jnp.zeros((_SPM_ROWS, _D), jnp.float32)

    es = [
        _edge_mlp(edge_attr, W1_0, b1_0.reshape(1, -1), W2_0, b2_0.reshape(1, -1)),
        _edge_mlp(edge_attr, W1_1, b1_1.reshape(1, -1), W2_1, b2_1.reshape(1, -1)),
        _edge_mlp(edge_attr, W1_2, b1_2.reshape(1, -1), W2_2, b2_2.reshape(1, -1)),
    ]

    h = x
    for e in es:
        agg = _sc_pass_fn()(h, src, dst, e, zeros)
        h = _update(h, agg)

    out = _pool(h, batch.reshape(_N // _BN, 1, _BN), Wl, bl.reshape(1, 1))
    return jnp.squeeze(out, -1)
